# baseline (speedup 1.0000x reference)
import jax
import jax.numpy as jnp
from jax.experimental import pallas as pl

_BM = 1024


def _gate_kernel(x_ref, w_ref, o_ref):
    x = x_ref[...].astype(jnp.bfloat16)
    o_ref[...] = jax.lax.dot_general(
        x, w_ref[...], (((1,), (1,)), ((), ())),
        preferred_element_type=jnp.float32)


def kernel(hidden_states, weight):
    m, k = hidden_states.shape
    e = weight.shape[0]
    w16 = weight.astype(jnp.bfloat16)
    return pl.pallas_call(
        _gate_kernel,
        grid=(m // _BM,),
        in_specs=[
            pl.BlockSpec((_BM, k), lambda i: (i, 0)),
            pl.BlockSpec((e, k), lambda i: (0, 0)),
        ],
        out_specs=pl.BlockSpec((_BM, e), lambda i: (i, 0)),
        out_shape=jax.ShapeDtypeStruct((m, e), jnp.float32),
    )(hidden_states, w16)


# speedup vs baseline: 1.0348x; 1.0348x over previous
import jax
import jax.numpy as jnp
from jax.experimental import pallas as pl

_BM = 512


def _gate_kernel(x_ref, w_ref, o_ref):
    o_ref[...] = jax.lax.dot_general(
        x_ref[...], w_ref[...], (((1,), (1,)), ((), ())),
        precision=jax.lax.Precision.DEFAULT,
        preferred_element_type=jnp.float32)


def kernel(hidden_states, weight):
    m, k = hidden_states.shape
    e = weight.shape[0]
    return pl.pallas_call(
        _gate_kernel,
        grid=(m // _BM,),
        in_specs=[
            pl.BlockSpec((_BM, k), lambda i: (i, 0)),
            pl.BlockSpec((e, k), lambda i: (0, 0)),
        ],
        out_specs=pl.BlockSpec((_BM, e), lambda i: (i, 0)),
        out_shape=jax.ShapeDtypeStruct((m, e), jnp.float32),
    )(hidden_states, weight)
